# initial kernel scaffold (unmeasured)
import jax
import jax.numpy as jnp
from jax import lax
from jax.experimental import pallas as pl
from jax.experimental.pallas import tpu as pltpu

N_DEV = 8


def _gelu(y):
    c = 0.7978845608028654
    return 0.5 * y * (1.0 + jnp.tanh(c * (y + 0.044715 * y * y * y)))


def kernel(x, w_mat):
    m_per, k = x.shape
    n = w_mat.shape[1]
    n_per = n // N_DEV
    m_total = N_DEV * m_per

    def body(x_ref, w_hbm, out_ref, w_buf, p_buf, w_sems, send_sems, recv_sems):
        my = lax.axis_index("i")

        barrier = pltpu.get_barrier_semaphore()
        for d in range(1, N_DEV):
            pl.semaphore_signal(
                barrier, inc=1,
                device_id=((my + d) % N_DEV,),
                device_id_type=pl.DeviceIdType.MESH,
            )
        pl.semaphore_wait(barrier, N_DEV - 1)

        def w_dma(s):
            c = (my + s) % N_DEV
            return pltpu.make_async_copy(
                w_hbm.at[:, pl.ds(c * n_per, n_per)],
                w_buf.at[s % 2],
                w_sems.at[s % 2],
            )

        w_dma(0).start()
        for s in range(N_DEV):
            if s + 1 < N_DEV:
                w_dma(s + 1).start()
            w_dma(s).wait()
            acc = jnp.dot(x_ref[:, :], w_buf[s % 2],
                          preferred_element_type=jnp.float32)
            if s == 0:
                out_ref[pl.ds(my * m_per, m_per), :] = acc
            else:
                p_buf[s] = acc
                tgt = (my + s) % N_DEV
                rdma = pltpu.make_async_remote_copy(
                    src_ref=p_buf.at[s],
                    dst_ref=out_ref.at[pl.ds(my * m_per, m_per)],
                    send_sem=send_sems.at[s],
                    recv_sem=recv_sems.at[s],
                    device_id=(tgt,),
                    device_id_type=pl.DeviceIdType.MESH,
                )
                rdma.start()

        for s in range(1, N_DEV):
            src_dev = (my + N_DEV - s) % N_DEV
            recv = pltpu.make_async_remote_copy(
                src_ref=p_buf.at[s],
                dst_ref=out_ref.at[pl.ds(src_dev * m_per, m_per)],
                send_sem=send_sems.at[s],
                recv_sem=recv_sems.at[s],
                device_id=(0,),
                device_id_type=pl.DeviceIdType.MESH,
            )
            recv.wait_recv()

        for s in range(1, N_DEV):
            send = pltpu.make_async_remote_copy(
                src_ref=p_buf.at[s],
                dst_ref=out_ref.at[pl.ds(0, m_per)],
                send_sem=send_sems.at[s],
                recv_sem=recv_sems.at[s],
                device_id=(0,),
                device_id_type=pl.DeviceIdType.MESH,
            )
            send.wait_send()

        out_ref[:, :] = _gelu(out_ref[:, :])

    return pl.pallas_call(
        body,
        out_shape=jax.ShapeDtypeStruct((m_total, n_per), jnp.float32),
        in_specs=[
            pl.BlockSpec(memory_space=pltpu.VMEM),
            pl.BlockSpec(memory_space=pltpu.ANY),
        ],
        out_specs=pl.BlockSpec(memory_space=pltpu.VMEM),
        scratch_shapes=[
            pltpu.VMEM((2, k, n_per), jnp.float32),
            pltpu.VMEM((N_DEV, m_per, n_per), jnp.float32),
            pltpu.SemaphoreType.DMA((2,)),
            pltpu.SemaphoreType.DMA((N_DEV,)),
            pltpu.SemaphoreType.DMA((N_DEV,)),
        ],
        compiler_params=pltpu.CompilerParams(collective_id=0),
    )(x, w_mat)


# baseline (device time: 176427 ns/iter reference)
import jax
import jax.numpy as jnp
from jax import lax
from jax.experimental import pallas as pl
from jax.experimental.pallas import tpu as pltpu

N_DEV = 8
N_SLOTS = 4


def _gelu(y):
    c = 0.7978845608028654
    return 0.5 * y * (1.0 + jnp.tanh(c * (y + 0.044715 * y * y * y)))


def kernel(x, w_mat):
    m_per, k = x.shape
    n = w_mat.shape[1]
    n_per = n // N_DEV
    m_total = N_DEV * m_per

    def body(x_ref, w_hbm, out_ref, w_buf, p_buf, w_sems, send_sems, recv_sems):
        my = lax.axis_index("i")

        barrier = pltpu.get_barrier_semaphore()
        for d in range(1, N_DEV):
            pl.semaphore_signal(
                barrier, inc=1,
                device_id=((my + d) % N_DEV,),
                device_id_type=pl.DeviceIdType.MESH,
            )
        pl.semaphore_wait(barrier, N_DEV - 1)

        k_chunk = k // 2
        n_chunks = N_DEV * 2

        def w_dma(t):
            s, h = divmod(t, 2)
            c = (my + s) % N_DEV
            return pltpu.make_async_copy(
                w_hbm.at[pl.ds(h * k_chunk, k_chunk), pl.ds(c * n_per, n_per)],
                w_buf.at[t % 2],
                w_sems.at[t % 2],
            )

        w_dma(0).start()
        for s in range(N_DEV):
            acc = None
            for h in range(2):
                t = s * 2 + h
                if t + 1 < n_chunks:
                    w_dma(t + 1).start()
                w_dma(t).wait()
                part = jnp.dot(x_ref[:, h * k_chunk:(h + 1) * k_chunk],
                               w_buf[t % 2],
                               preferred_element_type=jnp.float32)
                acc = part if acc is None else acc + part
            if s == 0:
                out_ref[pl.ds(my * m_per, m_per), :] = _gelu(acc)
            else:
                slot = s % N_SLOTS
                if s >= N_SLOTS + 1:
                    pltpu.make_async_remote_copy(
                        src_ref=p_buf.at[(s - N_SLOTS) % N_SLOTS],
                        dst_ref=out_ref.at[pl.ds(0, m_per)],
                        send_sem=send_sems.at[s - N_SLOTS],
                        recv_sem=recv_sems.at[s - N_SLOTS],
                        device_id=(0,),
                        device_id_type=pl.DeviceIdType.MESH,
                    ).wait_send()
                p_buf[slot] = acc
                tgt = (my + s) % N_DEV
                rdma = pltpu.make_async_remote_copy(
                    src_ref=p_buf.at[slot],
                    dst_ref=out_ref.at[pl.ds(my * m_per, m_per)],
                    send_sem=send_sems.at[s],
                    recv_sem=recv_sems.at[s],
                    device_id=(tgt,),
                    device_id_type=pl.DeviceIdType.MESH,
                )
                rdma.start()

        for s in range(1, N_DEV):
            src_dev = (my + N_DEV - s) % N_DEV
            rows = pl.ds(src_dev * m_per, m_per)
            recv = pltpu.make_async_remote_copy(
                src_ref=p_buf.at[s % N_SLOTS],
                dst_ref=out_ref.at[rows],
                send_sem=send_sems.at[s],
                recv_sem=recv_sems.at[s],
                device_id=(0,),
                device_id_type=pl.DeviceIdType.MESH,
            )
            recv.wait_recv()
            out_ref[rows, :] = _gelu(out_ref[rows, :])

        for s in range(max(1, N_DEV - N_SLOTS), N_DEV):
            send = pltpu.make_async_remote_copy(
                src_ref=p_buf.at[s % N_SLOTS],
                dst_ref=out_ref.at[pl.ds(0, m_per)],
                send_sem=send_sems.at[s],
                recv_sem=recv_sems.at[s],
                device_id=(0,),
                device_id_type=pl.DeviceIdType.MESH,
            )
            send.wait_send()

    return pl.pallas_call(
        body,
        out_shape=jax.ShapeDtypeStruct((m_total, n_per), jnp.float32),
        in_specs=[
            pl.BlockSpec(memory_space=pltpu.VMEM),
            pl.BlockSpec(memory_space=pl.ANY),
        ],
        out_specs=pl.BlockSpec(memory_space=pltpu.VMEM),
        scratch_shapes=[
            pltpu.VMEM((2, k // 2, n_per), jnp.float32),
            pltpu.VMEM((N_SLOTS, m_per, n_per), jnp.float32),
            pltpu.SemaphoreType.DMA((2,)),
            pltpu.SemaphoreType.DMA((N_DEV,)),
            pltpu.SemaphoreType.DMA((N_DEV,)),
        ],
        compiler_params=pltpu.CompilerParams(
            collective_id=0,
            vmem_limit_bytes=63 * 1024 * 1024,
        ),
    )(x, w_mat)


# device time: 164298 ns/iter; 1.0738x vs baseline; 1.0738x over previous
import jax
import jax.numpy as jnp
from jax import lax
from jax.experimental import pallas as pl
from jax.experimental.pallas import tpu as pltpu

N_DEV = 8
N_SLOTS = 4

def _partner(my, mask):
    z = my // 4
    q = my % 4
    x = jnp.where((q == 1) | (q == 2), 1, 0)
    y = jnp.where(q >= 2, 1, 0)
    x = x ^ (mask & 1)
    y = y ^ ((mask >> 1) & 1)
    z = z ^ ((mask >> 2) & 1)
    q2 = x * (1 - y) + 2 * x * y + 3 * (1 - x) * y
    return z * 4 + q2


def _gelu(y):
    c = 0.7978845608028654
    return 0.5 * y * (1.0 + jnp.tanh(c * (y + 0.044715 * y * y * y)))


def kernel(x, w_mat):
    m_per, k = x.shape
    n = w_mat.shape[1]
    n_per = n // N_DEV
    m_total = N_DEV * m_per

    def body(x_ref, w_hbm, out_ref, w_buf, p_buf, w_sems, send_sems, recv_sems):
        my = lax.axis_index("i")

        k_chunk = k // 2
        n_chunks = N_DEV * 2

        def w_dma(t):
            s, h = divmod(t, 2)
            c = _partner(my, s)
            return pltpu.make_async_copy(
                w_hbm.at[pl.ds(h * k_chunk, k_chunk), pl.ds(c * n_per, n_per)],
                w_buf.at[t % 2],
                w_sems.at[t % 2],
            )

        w_dma(0).start()

        barrier = pltpu.get_barrier_semaphore()
        for d in range(1, N_DEV):
            pl.semaphore_signal(
                barrier, inc=1,
                device_id=((my + d) % N_DEV,),
                device_id_type=pl.DeviceIdType.MESH,
            )
        pl.semaphore_wait(barrier, N_DEV - 1)

        for s in range(N_DEV):
            acc = None
            for h in range(2):
                t = s * 2 + h
                if t + 1 < n_chunks:
                    w_dma(t + 1).start()
                w_dma(t).wait()
                part = jnp.dot(x_ref[:, h * k_chunk:(h + 1) * k_chunk],
                               w_buf[t % 2],
                               preferred_element_type=jnp.float32)
                acc = part if acc is None else acc + part
            if s == 0:
                out_ref[pl.ds(my * m_per, m_per), :] = _gelu(acc)
            else:
                slot = s % N_SLOTS
                if s >= N_SLOTS + 1:
                    pltpu.make_async_remote_copy(
                        src_ref=p_buf.at[(s - N_SLOTS) % N_SLOTS],
                        dst_ref=out_ref.at[pl.ds(0, m_per)],
                        send_sem=send_sems.at[s - N_SLOTS],
                        recv_sem=recv_sems.at[s - N_SLOTS],
                        device_id=(0,),
                        device_id_type=pl.DeviceIdType.MESH,
                    ).wait_send()
                p_buf[slot] = acc
                tgt = _partner(my, s)
                rdma = pltpu.make_async_remote_copy(
                    src_ref=p_buf.at[slot],
                    dst_ref=out_ref.at[pl.ds(my * m_per, m_per)],
                    send_sem=send_sems.at[s],
                    recv_sem=recv_sems.at[s],
                    device_id=(tgt,),
                    device_id_type=pl.DeviceIdType.MESH,
                )
                rdma.start()

        for s in range(1, N_DEV):
            src_dev = _partner(my, s)
            rows = pl.ds(src_dev * m_per, m_per)
            recv = pltpu.make_async_remote_copy(
                src_ref=p_buf.at[s % N_SLOTS],
                dst_ref=out_ref.at[rows],
                send_sem=send_sems.at[s],
                recv_sem=recv_sems.at[s],
                device_id=(0,),
                device_id_type=pl.DeviceIdType.MESH,
            )
            recv.wait_recv()
            out_ref[rows, :] = _gelu(out_ref[rows, :])

        for s in range(max(1, N_DEV - N_SLOTS), N_DEV):
            send = pltpu.make_async_remote_copy(
                src_ref=p_buf.at[s % N_SLOTS],
                dst_ref=out_ref.at[pl.ds(0, m_per)],
                send_sem=send_sems.at[s],
                recv_sem=recv_sems.at[s],
                device_id=(0,),
                device_id_type=pl.DeviceIdType.MESH,
            )
            send.wait_send()

    return pl.pallas_call(
        body,
        out_shape=jax.ShapeDtypeStruct((m_total, n_per), jnp.float32),
        in_specs=[
            pl.BlockSpec(memory_space=pltpu.VMEM),
            pl.BlockSpec(memory_space=pl.ANY),
        ],
        out_specs=pl.BlockSpec(memory_space=pltpu.VMEM),
        scratch_shapes=[
            pltpu.VMEM((2, k // 2, n_per), jnp.float32),
            pltpu.VMEM((N_SLOTS, m_per, n_per), jnp.float32),
            pltpu.SemaphoreType.DMA((2,)),
            pltpu.SemaphoreType.DMA((N_DEV,)),
            pltpu.SemaphoreType.DMA((N_DEV,)),
        ],
        compiler_params=pltpu.CompilerParams(
            collective_id=0,
            vmem_limit_bytes=63 * 1024 * 1024,
        ),
    )(x, w_mat)


# device time: 119873 ns/iter; 1.4718x vs baseline; 1.3706x over previous
import jax
import jax.numpy as jnp
from jax import lax
from jax.experimental import pallas as pl
from jax.experimental.pallas import tpu as pltpu

N_DEV = 8
N_SLOTS = 3

def _partner(my, mask):
    z = my // 4
    q = my % 4
    x = jnp.where((q == 1) | (q == 2), 1, 0)
    y = jnp.where(q >= 2, 1, 0)
    x = x ^ (mask & 1)
    y = y ^ ((mask >> 1) & 1)
    z = z ^ ((mask >> 2) & 1)
    q2 = x * (1 - y) + 2 * x * y + 3 * (1 - x) * y
    return z * 4 + q2


def _gelu(y):
    c = 0.7978845608028654
    return 0.5 * y * (1.0 + jnp.tanh(c * (y + 0.044715 * y * y * y)))


def kernel(x, w_mat):
    m_per, k = x.shape
    n = w_mat.shape[1]
    n_per = n // N_DEV
    m_total = N_DEV * m_per

    def body(x_ref, w_hbm, out_ref, w_buf, p_buf, r_buf,
             w_sems, send_sems, recv_sems):
        my = lax.axis_index("i")

        k_chunk = k // 2
        n_chunks = N_DEV * 2

        def w_dma(t):
            s, h = divmod(t, 2)
            c = _partner(my, s)
            return pltpu.make_async_copy(
                w_hbm.at[pl.ds(h * k_chunk, k_chunk), pl.ds(c * n_per, n_per)],
                w_buf.at[t % 2],
                w_sems.at[t % 2],
            )

        w_dma(0).start()

        barrier = pltpu.get_barrier_semaphore()
        for d in range(1, N_DEV):
            pl.semaphore_signal(
                barrier, inc=1,
                device_id=((my + d) % N_DEV,),
                device_id_type=pl.DeviceIdType.MESH,
            )
        pl.semaphore_wait(barrier, N_DEV - 1)

        for s in range(N_DEV):
            acc = None
            for h in range(2):
                t = s * 2 + h
                if t + 1 < n_chunks:
                    w_dma(t + 1).start()
                w_dma(t).wait()
                part = jnp.dot(x_ref[:, h * k_chunk:(h + 1) * k_chunk],
                               w_buf[t % 2],
                               preferred_element_type=jnp.float32)
                acc = part if acc is None else acc + part
            if s == 0:
                out_ref[pl.ds(my * m_per, m_per), :] = _gelu(acc)
            else:
                slot = s % N_SLOTS
                if s >= N_SLOTS + 1:
                    pltpu.make_async_remote_copy(
                        src_ref=p_buf.at[(s - N_SLOTS) % N_SLOTS],
                        dst_ref=r_buf.at[0],
                        send_sem=send_sems.at[s - N_SLOTS],
                        recv_sem=recv_sems.at[s - N_SLOTS],
                        device_id=(0,),
                        device_id_type=pl.DeviceIdType.MESH,
                    ).wait_send()
                p_buf[slot] = acc.astype(jnp.bfloat16)
                tgt = _partner(my, s)
                rdma = pltpu.make_async_remote_copy(
                    src_ref=p_buf.at[slot],
                    dst_ref=r_buf.at[s - 1],
                    send_sem=send_sems.at[s],
                    recv_sem=recv_sems.at[s],
                    device_id=(tgt,),
                    device_id_type=pl.DeviceIdType.MESH,
                )
                rdma.start()

        for s in range(1, N_DEV):
            src_dev = _partner(my, s)
            rows = pl.ds(src_dev * m_per, m_per)
            recv = pltpu.make_async_remote_copy(
                src_ref=p_buf.at[s % N_SLOTS],
                dst_ref=r_buf.at[s - 1],
                send_sem=send_sems.at[s],
                recv_sem=recv_sems.at[s],
                device_id=(0,),
                device_id_type=pl.DeviceIdType.MESH,
            )
            recv.wait_recv()
            out_ref[rows, :] = _gelu(r_buf[s - 1].astype(jnp.float32))

        for s in range(max(1, N_DEV - N_SLOTS), N_DEV):
            send = pltpu.make_async_remote_copy(
                src_ref=p_buf.at[s % N_SLOTS],
                dst_ref=r_buf.at[0],
                send_sem=send_sems.at[s],
                recv_sem=recv_sems.at[s],
                device_id=(0,),
                device_id_type=pl.DeviceIdType.MESH,
            )
            send.wait_send()

    return pl.pallas_call(
        body,
        out_shape=jax.ShapeDtypeStruct((m_total, n_per), jnp.float32),
        in_specs=[
            pl.BlockSpec(memory_space=pltpu.VMEM),
            pl.BlockSpec(memory_space=pl.ANY),
        ],
        out_specs=pl.BlockSpec(memory_space=pltpu.VMEM),
        scratch_shapes=[
            pltpu.VMEM((2, k // 2, n_per), jnp.float32),
            pltpu.VMEM((N_SLOTS, m_per, n_per), jnp.bfloat16),
            pltpu.VMEM((N_DEV - 1, m_per, n_per), jnp.bfloat16),
            pltpu.SemaphoreType.DMA((2,)),
            pltpu.SemaphoreType.DMA((N_DEV,)),
            pltpu.SemaphoreType.DMA((N_DEV,)),
        ],
        compiler_params=pltpu.CompilerParams(
            collective_id=0,
            vmem_limit_bytes=63 * 1024 * 1024,
        ),
    )(x, w_mat)


# device time: 98623 ns/iter; 1.7889x vs baseline; 1.2155x over previous
import jax
import jax.numpy as jnp
from jax import lax
from jax.experimental import pallas as pl
from jax.experimental.pallas import tpu as pltpu

N_DEV = 8
N_SLOTS = 4
_MASKS = (1, 2, 4, 3, 6, 5, 7, 0)
_RECV_LAG = 3


def _partner(my, mask):
    z = my // 4
    q = my % 4
    x = jnp.where((q == 1) | (q == 2), 1, 0)
    y = jnp.where(q >= 2, 1, 0)
    x = x ^ (mask & 1)
    y = y ^ ((mask >> 1) & 1)
    z = z ^ ((mask >> 2) & 1)
    q2 = x * (1 - y) + 2 * x * y + 3 * (1 - x) * y
    return z * 4 + q2


def _gelu(y):
    c = 0.7978845608028654
    return 0.5 * y * (1.0 + jnp.tanh(c * (y + 0.044715 * y * y * y)))


def kernel(x, w_mat):
    m_per, k = x.shape
    n = w_mat.shape[1]
    n_per = n // N_DEV
    n_half = n_per // 2
    m_total = N_DEV * m_per
    n_chunks = 2 * N_DEV
    n_sends = 2 * (N_DEV - 1)

    def body(x_ref, w_hbm, out_ref, w_buf, p_buf, r_buf,
             w_sems, send_sems, recv_sems):
        my = lax.axis_index("i")

        def w_dma(t):
            j, h = divmod(t, 2)
            c = _partner(my, _MASKS[j])
            return pltpu.make_async_copy(
                w_hbm.at[:, pl.ds(c * n_per + h * n_half, n_half)],
                w_buf.at[t % 2],
                w_sems.at[t % 2],
            )

        def recv_chunk(t):
            j, h = divmod(t, 2)
            src_dev = _partner(my, _MASKS[j])
            pltpu.make_async_remote_copy(
                src_ref=p_buf.at[t % N_SLOTS],
                dst_ref=r_buf.at[j, h],
                send_sem=send_sems.at[t],
                recv_sem=recv_sems.at[t],
                device_id=(0,),
                device_id_type=pl.DeviceIdType.MESH,
            ).wait_recv()
            out_ref[pl.ds(src_dev * m_per, m_per),
                    h * n_half:(h + 1) * n_half] = _gelu(
                r_buf[j, h].astype(jnp.float32))

        w_dma(0).start()

        barrier = pltpu.get_barrier_semaphore()
        for d in range(1, N_DEV):
            pl.semaphore_signal(
                barrier, inc=1,
                device_id=((my + d) % N_DEV,),
                device_id_type=pl.DeviceIdType.MESH,
            )
        pl.semaphore_wait(barrier, N_DEV - 1)

        for t in range(n_chunks):
            j, h = divmod(t, 2)
            if t + 1 < n_chunks:
                w_dma(t + 1).start()
            w_dma(t).wait()
            part = jnp.dot(x_ref[:, :], w_buf[t % 2],
                           preferred_element_type=jnp.float32)
            if t < n_sends:
                slot = t % N_SLOTS
                if t >= N_SLOTS:
                    pltpu.make_async_remote_copy(
                        src_ref=p_buf.at[slot],
                        dst_ref=r_buf.at[0, 0],
                        send_sem=send_sems.at[t - N_SLOTS],
                        recv_sem=recv_sems.at[t - N_SLOTS],
                        device_id=(0,),
                        device_id_type=pl.DeviceIdType.MESH,
                    ).wait_send()
                p_buf[slot] = part.astype(jnp.bfloat16)
                pltpu.make_async_remote_copy(
                    src_ref=p_buf.at[slot],
                    dst_ref=r_buf.at[j, h],
                    send_sem=send_sems.at[t],
                    recv_sem=recv_sems.at[t],
                    device_id=(_partner(my, _MASKS[j]),),
                    device_id_type=pl.DeviceIdType.MESH,
                ).start()
            else:
                out_ref[pl.ds(my * m_per, m_per),
                        h * n_half:(h + 1) * n_half] = _gelu(part)
            if t >= _RECV_LAG and t - _RECV_LAG < n_sends:
                recv_chunk(t - _RECV_LAG)

        for t in range(max(0, n_chunks - _RECV_LAG), n_sends):
            recv_chunk(t)

        for t in range(max(0, n_sends - N_SLOTS), n_sends):
            pltpu.make_async_remote_copy(
                src_ref=p_buf.at[t % N_SLOTS],
                dst_ref=r_buf.at[0, 0],
                send_sem=send_sems.at[t],
                recv_sem=recv_sems.at[t],
                device_id=(0,),
                device_id_type=pl.DeviceIdType.MESH,
            ).wait_send()

    return pl.pallas_call(
        body,
        out_shape=jax.ShapeDtypeStruct((m_total, n_per), jnp.float32),
        in_specs=[
            pl.BlockSpec(memory_space=pltpu.VMEM),
            pl.BlockSpec(memory_space=pl.ANY),
        ],
        out_specs=pl.BlockSpec(memory_space=pltpu.VMEM),
        scratch_shapes=[
            pltpu.VMEM((2, k, n_half), jnp.float32),
            pltpu.VMEM((N_SLOTS, m_per, n_half), jnp.bfloat16),
            pltpu.VMEM((N_DEV - 1, 2, m_per, n_half), jnp.bfloat16),
            pltpu.SemaphoreType.DMA((2,)),
            pltpu.SemaphoreType.DMA((n_sends,)),
            pltpu.SemaphoreType.DMA((n_sends,)),
        ],
        compiler_params=pltpu.CompilerParams(
            collective_id=0,
            vmem_limit_bytes=63 * 1024 * 1024,
        ),
    )(x, w_mat)
